# Initial kernel scaffold; baseline (speedup 1.0000x reference)
import jax
import jax.numpy as jnp
from jax.experimental import pallas as pl
from jax.experimental.pallas import tpu as pltpu

D_MODEL = 768
L1 = 2048
L2 = 2048
BATCH = 16
H = 256
LTOT = L1 + L2
NUM_CHUNKS = LTOT // H
EXT = L2 + L2 + LTOT - L2


def _body(lens1_ref, boxes1_ref, b2ext_ref, l1c_ref, l2c_ref, w_ref, bias_ref,
          out_ref, mask_ref):
    i = pl.program_id(0)
    j0 = i * H
    w = w_ref[...]
    bias = bias_ref[...]
    rowid = j0 + jax.lax.broadcasted_iota(jnp.int32, (H, 1), 0)
    for col in range(BATCH):
        a1 = lens1_ref[col]
        x1 = boxes1_ref[:, col, :]
        start = L2 + j0 - a1
        x2 = b2ext_ref[pl.ds(start, H), col, :]
        src = jnp.where(rowid < a1, x1, x2)
        proj = jnp.dot(src, w, preferred_element_type=jnp.float32) + bias
        val = jnp.where(rowid < a1 + L2, proj, 0.0)
        out_ref[:, col, :] = val
    flens = l1c_ref[...] + l2c_ref[...]
    colid = j0 + jax.lax.broadcasted_iota(jnp.int32, (BATCH, H), 1)
    mask_ref[...] = colid >= flens


def kernel(boxes1, lengths1, boxes2, lengths2, W, b):
    b2ext = jnp.pad(boxes2, ((L2, EXT - L2 - L2), (0, 0), (0, 0)))
    l1c = lengths1.reshape(BATCH, 1)
    l2c = lengths2.reshape(BATCH, 1)
    bias2d = b.reshape(1, D_MODEL)

    grid_spec = pltpu.PrefetchScalarGridSpec(
        num_scalar_prefetch=1,
        grid=(NUM_CHUNKS,),
        in_specs=[
            pl.BlockSpec((H, BATCH, 4), lambda i, s: (i, 0, 0)),
            pl.BlockSpec((EXT, BATCH, 4), lambda i, s: (0, 0, 0)),
            pl.BlockSpec((BATCH, 1), lambda i, s: (0, 0)),
            pl.BlockSpec((BATCH, 1), lambda i, s: (0, 0)),
            pl.BlockSpec((4, D_MODEL), lambda i, s: (0, 0)),
            pl.BlockSpec((1, D_MODEL), lambda i, s: (0, 0)),
        ],
        out_specs=[
            pl.BlockSpec((H, BATCH, D_MODEL), lambda i, s: (i, 0, 0)),
            pl.BlockSpec((BATCH, H), lambda i, s: (0, i)),
        ],
    )
    out, mask = pl.pallas_call(
        _body,
        grid_spec=grid_spec,
        out_shape=[
            jax.ShapeDtypeStruct((LTOT, BATCH, D_MODEL), jnp.float32),
            jax.ShapeDtypeStruct((BATCH, LTOT), jnp.bool_),
        ],
        compiler_params=pltpu.CompilerParams(
            dimension_semantics=("arbitrary",),
        ),
    )(lengths1, boxes1, b2ext, l1c, l2c, W, bias2d)
    return out, mask


# speedup vs baseline: 3.6195x; 3.6195x over previous
import jax
import jax.numpy as jnp
from jax.experimental import pallas as pl
from jax.experimental.pallas import tpu as pltpu

D_MODEL = 768
L1 = 2048
L2 = 2048
BATCH = 16
H = 128
LTOT = L1 + L2
NUM_CHUNKS = LTOT // H
EXT = L2 + LTOT
DOUT = BATCH * D_MODEL


def _body(lens1_ref, b1_ref, b2_ref, l1lane_ref, l1rep_ref, l1c_ref, l2c_ref,
          wbd_ref, bias_ref, out_ref, mask_ref):
    i = pl.program_id(0)
    j0 = i * H
    rowid = j0 + jax.lax.broadcasted_iota(jnp.int32, (H, 1), 0)
    laneq = jax.lax.broadcasted_iota(jnp.int32, (H, 64), 1) // 4
    src2 = jnp.zeros((H, 64), jnp.float32)
    for col in range(BATCH):
        start = L2 + j0 - lens1_ref[col]
        win = b2_ref[pl.ds(start, H), :]
        src2 = jnp.where(laneq == col, win, src2)
    src = jnp.where(rowid < l1lane_ref[...], b1_ref[...], src2)
    res = jnp.dot(src, wbd_ref[...], preferred_element_type=jnp.float32)
    res = res + bias_ref[...]
    out_ref[...] = jnp.where(rowid < l1rep_ref[...] + L2, res, 0.0)
    flens = l1c_ref[...] + l2c_ref[...]
    colid = j0 + jax.lax.broadcasted_iota(jnp.int32, (BATCH, H), 1)
    mask_ref[...] = colid >= flens


def kernel(boxes1, lengths1, boxes2, lengths2, W, b):
    b1_flat = boxes1.reshape(L1, 64)
    b2_flat = jnp.pad(boxes2.reshape(L2, 64), ((L2, EXT - L2 - L2), (0, 0)))
    l1lane = jnp.repeat(lengths1, 4).reshape(1, 64)
    l1rep = jnp.repeat(lengths1, D_MODEL).reshape(1, DOUT)
    l1c = lengths1.reshape(BATCH, 1)
    l2c = lengths2.reshape(BATCH, 1)
    wbd = jnp.kron(jnp.eye(BATCH, dtype=W.dtype), W)
    bias_all = jnp.tile(b, BATCH).reshape(1, DOUT)

    grid_spec = pltpu.PrefetchScalarGridSpec(
        num_scalar_prefetch=1,
        grid=(NUM_CHUNKS,),
        in_specs=[
            pl.BlockSpec((H, 64), lambda i, s: (i, 0)),
            pl.BlockSpec((EXT, 64), lambda i, s: (0, 0)),
            pl.BlockSpec((1, 64), lambda i, s: (0, 0)),
            pl.BlockSpec((1, DOUT), lambda i, s: (0, 0)),
            pl.BlockSpec((BATCH, 1), lambda i, s: (0, 0)),
            pl.BlockSpec((BATCH, 1), lambda i, s: (0, 0)),
            pl.BlockSpec((64, DOUT), lambda i, s: (0, 0)),
            pl.BlockSpec((1, DOUT), lambda i, s: (0, 0)),
        ],
        out_specs=[
            pl.BlockSpec((H, DOUT), lambda i, s: (i, 0)),
            pl.BlockSpec((BATCH, H), lambda i, s: (0, i)),
        ],
    )
    out2d, mask = pl.pallas_call(
        _body,
        grid_spec=grid_spec,
        out_shape=[
            jax.ShapeDtypeStruct((LTOT, DOUT), jnp.float32),
            jax.ShapeDtypeStruct((BATCH, LTOT), jnp.bool_),
        ],
        compiler_params=pltpu.CompilerParams(
            dimension_semantics=("arbitrary",),
        ),
    )(lengths1, b1_flat, b2_flat, l1lane, l1rep, l1c, l2c, wbd, bias_all)
    return out2d.reshape(LTOT, BATCH, D_MODEL), mask


# speedup vs baseline: 8.1595x; 2.2544x over previous
import jax
import jax.numpy as jnp
from jax.experimental import pallas as pl
from jax.experimental.pallas import tpu as pltpu

D_MODEL = 768
L1 = 2048
L2 = 2048
BATCH = 16
H = 128
LTOT = L1 + L2
NUM_CHUNKS = LTOT // H
EXT = L2 + LTOT
DOUT = BATCH * D_MODEL


def _body(lens1_ref, b1_ref, b2_ref, l1lane_ref, l1rep_ref, l1c_ref, l2c_ref,
          wbd_ref, bias_ref, out_ref, mask_ref):
    i = pl.program_id(0)
    j0 = i * H
    rowid = j0 + jax.lax.broadcasted_iota(jnp.int32, (H, 1), 0)
    laneq = jax.lax.broadcasted_iota(jnp.int32, (H, 64), 1) // 4
    src2 = jnp.zeros((H, 64), jnp.float32)
    for col in range(BATCH):
        start = L2 + j0 - lens1_ref[col]
        win = b2_ref[pl.ds(start, H), :]
        src2 = jnp.where(laneq == col, win, src2)
    src = jnp.where(rowid < l1lane_ref[...], b1_ref[...], src2)
    res = jnp.dot(src, wbd_ref[...], preferred_element_type=jnp.float32)
    res = res + bias_ref[...]
    res = jnp.where(rowid < l1rep_ref[...] + L2, res, 0.0)
    out_ref[...] = res.reshape(H, BATCH, D_MODEL)
    flens = l1c_ref[...] + l2c_ref[...]
    colid = j0 + jax.lax.broadcasted_iota(jnp.int32, (BATCH, H), 1)
    mask_ref[...] = colid >= flens


def kernel(boxes1, lengths1, boxes2, lengths2, W, b):
    b1_flat = boxes1.reshape(L1, 64)
    b2_flat = jnp.pad(boxes2.reshape(L2, 64), ((L2, EXT - L2 - L2), (0, 0)))
    l1lane = jnp.repeat(lengths1, 4).reshape(1, 64)
    l1rep = jnp.repeat(lengths1, D_MODEL).reshape(1, DOUT)
    l1c = lengths1.reshape(BATCH, 1)
    l2c = lengths2.reshape(BATCH, 1)
    wbd = jnp.kron(jnp.eye(BATCH, dtype=W.dtype), W)
    bias_all = jnp.tile(b, BATCH).reshape(1, DOUT)

    grid_spec = pltpu.PrefetchScalarGridSpec(
        num_scalar_prefetch=1,
        grid=(NUM_CHUNKS,),
        in_specs=[
            pl.BlockSpec((H, 64), lambda i, s: (i, 0)),
            pl.BlockSpec((EXT, 64), lambda i, s: (0, 0)),
            pl.BlockSpec((1, 64), lambda i, s: (0, 0)),
            pl.BlockSpec((1, DOUT), lambda i, s: (0, 0)),
            pl.BlockSpec((BATCH, 1), lambda i, s: (0, 0)),
            pl.BlockSpec((BATCH, 1), lambda i, s: (0, 0)),
            pl.BlockSpec((64, DOUT), lambda i, s: (0, 0)),
            pl.BlockSpec((1, DOUT), lambda i, s: (0, 0)),
        ],
        out_specs=[
            pl.BlockSpec((H, BATCH, D_MODEL), lambda i, s: (i, 0, 0)),
            pl.BlockSpec((BATCH, H), lambda i, s: (0, i)),
        ],
    )
    out, mask = pl.pallas_call(
        _body,
        grid_spec=grid_spec,
        out_shape=[
            jax.ShapeDtypeStruct((LTOT, BATCH, D_MODEL), jnp.float32),
            jax.ShapeDtypeStruct((BATCH, LTOT), jnp.bool_),
        ],
        compiler_params=pltpu.CompilerParams(
            dimension_semantics=("arbitrary",),
        ),
    )(lengths1, b1_flat, b2_flat, l1lane, l1rep, l1c, l2c, wbd, bias_all)
    return out, mask


# speedup vs baseline: 8.4394x; 1.0343x over previous
import jax
import jax.numpy as jnp
from jax.experimental import pallas as pl
from jax.experimental.pallas import tpu as pltpu

D_MODEL = 768
L1 = 2048
L2 = 2048
BATCH = 16
NCOORD = 5
LANES = BATCH * NCOORD
H = 128
LTOT = L1 + L2
NUM_CHUNKS = LTOT // H
EXT = L2 + LTOT
DOUT = BATCH * D_MODEL


def _body(lens1_ref, b1_ref, b2_ref, l1lane_ref, l1c_ref, l2c_ref,
          wbd_ref, out_ref, mask_ref):
    i = pl.program_id(0)
    j0 = i * H
    rowid = j0 + jax.lax.broadcasted_iota(jnp.int32, (H, 1), 0)
    laneq = jax.lax.broadcasted_iota(jnp.int32, (H, LANES), 1) // NCOORD
    src2 = jnp.zeros((H, LANES), jnp.float32)
    for col in range(BATCH):
        start = L2 + j0 - lens1_ref[col]
        win = b2_ref[pl.ds(start, H), :]
        src2 = jnp.where(laneq == col, win, src2)
    src = jnp.where(rowid < l1lane_ref[...], b1_ref[...], src2)
    res = jnp.dot(src, wbd_ref[...], preferred_element_type=jnp.float32)
    out_ref[...] = res.reshape(H, BATCH, D_MODEL)
    flens = l1c_ref[...] + l2c_ref[...]
    colid = j0 + jax.lax.broadcasted_iota(jnp.int32, (BATCH, H), 1)
    mask_ref[...] = colid >= flens


def kernel(boxes1, lengths1, boxes2, lengths2, W, b):
    ones1 = jnp.ones((L1, BATCH, 1), jnp.float32)
    b1_flat = jnp.concatenate([boxes1, ones1], axis=2).reshape(L1, LANES)
    b2a = jnp.concatenate([boxes2, ones1], axis=2).reshape(L2, LANES)
    b2_flat = jnp.pad(b2a, ((L2, EXT - L2 - L2), (0, 0)))
    l1lane = jnp.repeat(lengths1, NCOORD).reshape(1, LANES)
    l1c = lengths1.reshape(BATCH, 1)
    l2c = lengths2.reshape(BATCH, 1)
    w5 = jnp.concatenate([W, b.reshape(1, D_MODEL)], axis=0)
    wbd = jnp.kron(jnp.eye(BATCH, dtype=W.dtype), w5)

    grid_spec = pltpu.PrefetchScalarGridSpec(
        num_scalar_prefetch=1,
        grid=(NUM_CHUNKS,),
        in_specs=[
            pl.BlockSpec((H, LANES), lambda i, s: (i, 0)),
            pl.BlockSpec((EXT, LANES), lambda i, s: (0, 0)),
            pl.BlockSpec((1, LANES), lambda i, s: (0, 0)),
            pl.BlockSpec((BATCH, 1), lambda i, s: (0, 0)),
            pl.BlockSpec((BATCH, 1), lambda i, s: (0, 0)),
            pl.BlockSpec((LANES, DOUT), lambda i, s: (0, 0)),
        ],
        out_specs=[
            pl.BlockSpec((H, BATCH, D_MODEL), lambda i, s: (i, 0, 0)),
            pl.BlockSpec((BATCH, H), lambda i, s: (0, i)),
        ],
    )
    out, mask = pl.pallas_call(
        _body,
        grid_spec=grid_spec,
        out_shape=[
            jax.ShapeDtypeStruct((LTOT, BATCH, D_MODEL), jnp.float32),
            jax.ShapeDtypeStruct((BATCH, LTOT), jnp.bool_),
        ],
        compiler_params=pltpu.CompilerParams(
            dimension_semantics=("arbitrary",),
        ),
    )(lengths1, b1_flat, b2_flat, l1lane, l1c, l2c, wbd)
    return out, mask


# speedup vs baseline: 8.9904x; 1.0653x over previous
import jax
import jax.numpy as jnp
from jax.experimental import pallas as pl
from jax.experimental.pallas import tpu as pltpu

D_MODEL = 768
L1 = 2048
L2 = 2048
BATCH = 16
NCOORD = 5
LANES = BATCH * NCOORD
H = 256
LTOT = L1 + L2
NUM_CHUNKS = LTOT // H
EXT = L2 + LTOT
DOUT = BATCH * D_MODEL


def _body(lens1_ref, b1_ref, b2_ref, l1lane_ref, l1c_ref, l2c_ref,
          wbd_ref, out_ref, mask_ref):
    i = pl.program_id(0)
    j0 = i * H
    rowid = j0 + jax.lax.broadcasted_iota(jnp.int32, (H, 1), 0)
    laneq = jax.lax.broadcasted_iota(jnp.int32, (H, LANES), 1) // NCOORD
    src2 = jnp.zeros((H, LANES), jnp.float32)
    for col in range(BATCH):
        start = L2 + j0 - lens1_ref[col]
        win = b2_ref[pl.ds(start, H), :]
        src2 = jnp.where(laneq == col, win, src2)
    src = jnp.where(rowid < l1lane_ref[...], b1_ref[...], src2)
    res = jnp.dot(src, wbd_ref[...], preferred_element_type=jnp.float32)
    out_ref[...] = res.reshape(H, BATCH, D_MODEL)
    flens = l1c_ref[...] + l2c_ref[...]
    colid = j0 + jax.lax.broadcasted_iota(jnp.int32, (BATCH, H), 1)
    mask_ref[...] = colid >= flens


def kernel(boxes1, lengths1, boxes2, lengths2, W, b):
    ones1 = jnp.ones((L1, BATCH, 1), jnp.float32)
    b1_flat = jnp.concatenate([boxes1, ones1], axis=2).reshape(L1, LANES)
    b2a = jnp.concatenate([boxes2, ones1], axis=2).reshape(L2, LANES)
    b2_flat = jnp.pad(b2a, ((L2, EXT - L2 - L2), (0, 0)))
    l1lane = jnp.repeat(lengths1, NCOORD).reshape(1, LANES)
    l1c = lengths1.reshape(BATCH, 1)
    l2c = lengths2.reshape(BATCH, 1)
    w5 = jnp.concatenate([W, b.reshape(1, D_MODEL)], axis=0)
    wbd = jnp.kron(jnp.eye(BATCH, dtype=W.dtype), w5)

    grid_spec = pltpu.PrefetchScalarGridSpec(
        num_scalar_prefetch=1,
        grid=(NUM_CHUNKS,),
        in_specs=[
            pl.BlockSpec((H, LANES), lambda i, s: (i, 0)),
            pl.BlockSpec((EXT, LANES), lambda i, s: (0, 0)),
            pl.BlockSpec((1, LANES), lambda i, s: (0, 0)),
            pl.BlockSpec((BATCH, 1), lambda i, s: (0, 0)),
            pl.BlockSpec((BATCH, 1), lambda i, s: (0, 0)),
            pl.BlockSpec((LANES, DOUT), lambda i, s: (0, 0)),
        ],
        out_specs=[
            pl.BlockSpec((H, BATCH, D_MODEL), lambda i, s: (i, 0, 0)),
            pl.BlockSpec((BATCH, H), lambda i, s: (0, i)),
        ],
    )
    out, mask = pl.pallas_call(
        _body,
        grid_spec=grid_spec,
        out_shape=[
            jax.ShapeDtypeStruct((LTOT, BATCH, D_MODEL), jnp.float32),
            jax.ShapeDtypeStruct((BATCH, LTOT), jnp.bool_),
        ],
        compiler_params=pltpu.CompilerParams(
            dimension_semantics=("arbitrary",),
        ),
    )(lengths1, b1_flat, b2_flat, l1lane, l1c, l2c, wbd)
    return out, mask
